# baseline (speedup 1.0000x reference)
import jax
import jax.numpy as jnp
from jax.experimental import pallas as pl
from jax.experimental.pallas import tpu as pltpu

_XBLK = 2048
_ZBLK = 1024


def _x_kernel(x_ref, y_ref, o_ref):
    i = pl.program_id(0)

    @pl.when(i != 0)
    def _plain():
        o_ref[...] = x_ref[...]

    @pl.when(i == 0)
    def _fixup():
        blk = x_ref[...]
        rows = jax.lax.broadcasted_iota(jnp.int32, blk.shape, 0)
        blk = jnp.where(rows == 10, y_ref[0:1, :], blk)
        blk = jnp.where(rows == 2, y_ref[1:2, :], blk)
        blk = jnp.where(rows == 1, jnp.float32(45.0), blk)
        o_ref[...] = blk


def _z_kernel(z_ref, w_ref, o_ref):
    i = pl.program_id(0)

    @pl.when(i != 0)
    def _plain():
        o_ref[...] = z_ref[...]

    @pl.when(i == 0)
    def _fixup():
        blk = z_ref[...]
        rows = jax.lax.broadcasted_iota(jnp.int32, blk.shape, 0)
        cols = jax.lax.broadcasted_iota(jnp.int32, blk.shape, 1)
        upd = jnp.where((rows == 1) & (cols == 3), w_ref[0], 0.0)
        upd = jnp.where((rows == 0) & (cols == 2), w_ref[1], upd)
        upd = jnp.where((rows == 0) & (cols == 1), w_ref[2], upd)
        o_ref[...] = blk + upd


def kernel(x, y, z, w):
    x_out = pl.pallas_call(
        _x_kernel,
        grid=(x.shape[0] // _XBLK,),
        in_specs=[
            pl.BlockSpec((_XBLK, x.shape[1]), lambda i: (i, 0)),
            pl.BlockSpec((2, x.shape[1]), lambda i: (0, 0)),
        ],
        out_specs=pl.BlockSpec((_XBLK, x.shape[1]), lambda i: (i, 0)),
        out_shape=jax.ShapeDtypeStruct(x.shape, x.dtype),
    )(x, y)
    z_out = pl.pallas_call(
        _z_kernel,
        grid=(z.shape[0] // _ZBLK,),
        in_specs=[
            pl.BlockSpec((_ZBLK, z.shape[1]), lambda i: (i, 0)),
            pl.BlockSpec(memory_space=pltpu.SMEM),
        ],
        out_specs=pl.BlockSpec((_ZBLK, z.shape[1]), lambda i: (i, 0)),
        out_shape=jax.ShapeDtypeStruct(z.shape, z.dtype),
    )(z, w)
    return (x_out, z_out)
